# baseline (speedup 1.0000x reference)
import functools

import jax
import jax.numpy as jnp
from jax import lax
from jax.experimental import pallas as pl
from jax.experimental.pallas import tpu as pltpu
from jax.experimental.pallas import tpu_sc as plsc

DIM = 64
EPS = 1e-05

NUM_CORES = 2
NUM_SUBCORES = 16
NW = NUM_CORES * NUM_SUBCORES

CHUNK = 128
FIRE = 16


def _make_gather(batch: int):
  assert batch % (NW * CHUNK) == 0
  b_per_w = batch // NW
  n_chunks = b_per_w // CHUNK
  mesh = plsc.VectorSubcoreMesh(
      core_axis_name="c", subcore_axis_name="s",
      num_cores=NUM_CORES, num_subcores=NUM_SUBCORES)

  @functools.partial(
      pl.kernel,
      out_type=jax.ShapeDtypeStruct((batch // 2, 2 * DIM), jnp.float32),
      mesh=mesh,
      scratch_types=[
          pltpu.VMEM((b_per_w,), jnp.int32),
          pltpu.VMEM((FIRE, 8, DIM), jnp.float32),
          pltpu.VMEM((CHUNK // 2, 2 * DIM), jnp.float32),
          pltpu.SemaphoreType.DMA,
      ],
  )
  def gather_kernel(idx_hbm, table_hbm, out_hbm, idx_v, tiles_v, rows_p, sem):
    wid = lax.axis_index("s") * NUM_CORES + lax.axis_index("c")
    base = pl.multiple_of(wid * b_per_w, CHUNK)
    pltpu.sync_copy(idx_hbm.at[pl.ds(base, b_per_w)], idx_v)

    @pl.loop(0, n_chunks)
    def _chunk(c):
      off = pl.multiple_of(c * CHUNK, CHUNK)

      @pl.loop(0, CHUNK // FIRE)
      def _round(rr):
        i0 = rr * FIRE
        rv = idx_v[pl.ds(off + i0, FIRE)]
        for u in range(FIRE):
          t8 = pl.multiple_of((rv[u] >> 3) * 8, 8)
          pltpu.async_copy(
              table_hbm.at[pl.ds(t8, 8)], tiles_v.at[u], sem)
        for u in range(FIRE):
          t8 = pl.multiple_of((rv[u] >> 3) * 8, 8)
          pltpu.make_async_copy(
              table_hbm.at[pl.ds(t8, 8)], tiles_v.at[u], sem).wait()
        for u in range(FIRE):
          sub = rv[u] & 7
          for j in range(DIM // 16):
            rows_p[(i0 + u) // 2,
                   pl.ds(DIM * (u % 2) + 16 * j, 16)] = (
                       tiles_v[u, sub, pl.ds(16 * j, 16)])

      pltpu.sync_copy(
          rows_p,
          out_hbm.at[pl.ds(
              pl.multiple_of((base + off) // 2, CHUNK // 2), CHUNK // 2)])

  return gather_kernel


def _ln_body(x_ref, s_ref, b_ref, o_ref):
  g, seq, _ = o_ref.shape
  x = x_ref[...].reshape(g, seq // 2, 2 * DIM)
  xb = jnp.broadcast_to(x[:, :, None, :], (g, seq // 2, 2, 2 * DIM))
  xb = xb.reshape(g, seq, 2 * DIM)
  odd = (lax.broadcasted_iota(jnp.int32, (1, seq, 1), 1) % 2) == 1
  xsel = jnp.where(odd, xb[:, :, DIM:], xb[:, :, :DIM])
  mean = jnp.mean(xsel, axis=-1, keepdims=True)
  var = jnp.mean(jnp.square(xsel - mean), axis=-1, keepdims=True)
  inv = s_ref[...] * lax.rsqrt(var + EPS)
  o_ref[...] = xsel * inv + (b_ref[...] - mean * inv)


def _make_ln(groups: int, seq: int, gblk: int):
  assert groups % gblk == 0 and seq % 2 == 0
  pblk = gblk * seq // 2
  return pl.pallas_call(
      _ln_body,
      grid=(groups // gblk,),
      in_specs=[
          pl.BlockSpec((pblk, 2 * DIM), lambda i: (i, 0)),
          pl.BlockSpec((1, 1, DIM), lambda i: (0, 0, 0)),
          pl.BlockSpec((1, 1, DIM), lambda i: (0, 0, 0)),
      ],
      out_specs=pl.BlockSpec((gblk, seq, DIM), lambda i: (i, 0, 0)),
      out_shape=jax.ShapeDtypeStruct((groups, seq, DIM), jnp.float32),
  )


def kernel(inputs, emb_weight, ln_scale, ln_bias):
  groups, seq = inputs.shape
  idx = jnp.asarray(inputs, jnp.int32).reshape(-1)
  packed = _make_gather(groups * seq)(idx, emb_weight)
  return _make_ln(groups, seq, 64)(
      packed, ln_scale.reshape(1, 1, DIM), ln_bias.reshape(1, 1, DIM))


# speedup vs baseline: 1.2939x; 1.2939x over previous
import functools

import jax
import jax.numpy as jnp
from jax import lax
from jax.experimental import pallas as pl
from jax.experimental.pallas import tpu as pltpu
from jax.experimental.pallas import tpu_sc as plsc

DIM = 64
EPS = 1e-05

NUM_CORES = 2
NUM_SUBCORES = 16
NW = NUM_CORES * NUM_SUBCORES

CHUNK = 128


def _make_gather(batch: int):
  assert batch % (NW * CHUNK) == 0
  b_per_w = batch // NW
  n_chunks = b_per_w // CHUNK
  mesh = plsc.VectorSubcoreMesh(
      core_axis_name="c", subcore_axis_name="s",
      num_cores=NUM_CORES, num_subcores=NUM_SUBCORES)

  @functools.partial(
      pl.kernel,
      out_type=jax.ShapeDtypeStruct((batch, 2 * DIM), jnp.float32),
      mesh=mesh,
      scratch_types=[
          pltpu.VMEM((b_per_w,), jnp.int32),
          pltpu.VMEM((CHUNK, 2 * DIM), jnp.float32),
          pltpu.SemaphoreType.DMA,
      ],
  )
  def gather_kernel(idx_hbm, table_hbm, out_hbm, idx_v, rows_v, sem):
    wid = lax.axis_index("s") * NUM_CORES + lax.axis_index("c")
    base = wid * b_per_w
    pltpu.sync_copy(idx_hbm.at[pl.ds(base, b_per_w)], idx_v)

    @pl.loop(0, n_chunks)
    def _chunk(c):
      off = c * CHUNK
      pltpu.async_copy(
          table_hbm.at[idx_v.at[pl.ds(off, CHUNK)]], rows_v, sem).wait()
      pltpu.sync_copy(rows_v, out_hbm.at[pl.ds(base + off, CHUNK)])

  return gather_kernel


def _ln_body(x_ref, pm_ref, s_ref, b_ref, o_ref):
  g, seq, _ = o_ref.shape
  x = x_ref[...].reshape(g, seq, 2 * DIM)
  pm = pm_ref[...].reshape(g, seq, 1)
  xsel = jnp.where(pm > 0.5, x[:, :, DIM:], x[:, :, :DIM])
  mean = jnp.mean(xsel, axis=-1, keepdims=True)
  var = jnp.mean(jnp.square(xsel - mean), axis=-1, keepdims=True)
  inv = s_ref[...] * lax.rsqrt(var + EPS)
  o_ref[...] = xsel * inv + (b_ref[...] - mean * inv)


def _make_ln(groups: int, seq: int, gblk: int):
  assert groups % gblk == 0
  rblk = gblk * seq
  return pl.pallas_call(
      _ln_body,
      grid=(groups // gblk,),
      in_specs=[
          pl.BlockSpec((rblk, 2 * DIM), lambda i: (i, 0)),
          pl.BlockSpec((gblk, seq), lambda i: (i, 0)),
          pl.BlockSpec((1, 1, DIM), lambda i: (0, 0, 0)),
          pl.BlockSpec((1, 1, DIM), lambda i: (0, 0, 0)),
      ],
      out_specs=pl.BlockSpec((gblk, seq, DIM), lambda i: (i, 0, 0)),
      out_shape=jax.ShapeDtypeStruct((groups, seq, DIM), jnp.float32),
  )


def kernel(inputs, emb_weight, ln_scale, ln_bias):
  groups, seq = inputs.shape
  idx = jnp.asarray(inputs, jnp.int32)
  pair = (idx >> 1).reshape(-1)
  pmask = (idx & 1).astype(jnp.float32)
  table_pairs = emb_weight.reshape(-1, 2 * DIM)
  packed = _make_gather(groups * seq)(pair, table_pairs)
  return _make_ln(groups, seq, 128)(
      packed, pmask, ln_scale.reshape(1, 1, DIM), ln_bias.reshape(1, 1, DIM))


# speedup vs baseline: 1.3066x; 1.0098x over previous
import functools

import jax
import jax.numpy as jnp
from jax import lax
from jax.experimental import pallas as pl
from jax.experimental.pallas import tpu as pltpu
from jax.experimental.pallas import tpu_sc as plsc

DIM = 64
EPS = 1e-05

NUM_CORES = 2
NUM_SUBCORES = 16
NW = NUM_CORES * NUM_SUBCORES

CHUNK = 128


def _make_gather(batch: int):
  assert batch % (NW * CHUNK) == 0
  b_per_w = batch // NW
  n_chunks = b_per_w // CHUNK
  mesh = plsc.VectorSubcoreMesh(
      core_axis_name="c", subcore_axis_name="s",
      num_cores=NUM_CORES, num_subcores=NUM_SUBCORES)

  @functools.partial(
      pl.kernel,
      out_type=jax.ShapeDtypeStruct((batch, 2 * DIM), jnp.float32),
      mesh=mesh,
      scratch_types=[
          pltpu.VMEM((b_per_w,), jnp.int32),
          pltpu.VMEM((CHUNK, 2 * DIM), jnp.float32),
          pltpu.SemaphoreType.DMA,
      ],
  )
  def gather_kernel(idx_hbm, table_hbm, out_hbm, idx_v, rows_v, sem):
    wid = lax.axis_index("s") * NUM_CORES + lax.axis_index("c")
    base = wid * b_per_w
    pltpu.sync_copy(idx_hbm.at[pl.ds(base, b_per_w)], idx_v)

    @pl.loop(0, n_chunks)
    def _chunk(c):
      off = c * CHUNK
      pltpu.async_copy(
          table_hbm.at[idx_v.at[pl.ds(off, CHUNK)]], rows_v, sem).wait()
      pltpu.sync_copy(rows_v, out_hbm.at[pl.ds(base + off, CHUNK)])

  return gather_kernel


def _ln_body(x_ref, s_ref, b_ref, o_ref):
  g, seq, _ = o_ref.shape
  x = x_ref[...].reshape(g, seq, 2 * DIM)[:, :, :DIM]
  mean = jnp.mean(x, axis=-1, keepdims=True)
  var = jnp.mean(jnp.square(x - mean), axis=-1, keepdims=True)
  inv = s_ref[...] * lax.rsqrt(var + EPS)
  o_ref[...] = x * inv + (b_ref[...] - mean * inv)


def _make_ln(groups: int, seq: int, gblk: int):
  assert groups % gblk == 0
  rblk = gblk * seq
  return pl.pallas_call(
      _ln_body,
      grid=(groups // gblk,),
      in_specs=[
          pl.BlockSpec((rblk, 2 * DIM), lambda i: (i, 0)),
          pl.BlockSpec((1, 1, DIM), lambda i: (0, 0, 0)),
          pl.BlockSpec((1, 1, DIM), lambda i: (0, 0, 0)),
      ],
      out_specs=pl.BlockSpec((gblk, seq, DIM), lambda i: (i, 0, 0)),
      out_shape=jax.ShapeDtypeStruct((groups, seq, DIM), jnp.float32),
  )


def kernel(inputs, emb_weight, ln_scale, ln_bias):
  groups, seq = inputs.shape
  idx = jnp.asarray(inputs, jnp.int32).reshape(-1)
  table_pad = jnp.pad(emb_weight, ((0, 0), (0, DIM)))
  packed = _make_gather(groups * seq)(idx, table_pad)
  return _make_ln(groups, seq, 64)(
      packed, ln_scale.reshape(1, 1, DIM), ln_bias.reshape(1, 1, DIM))


# speedup vs baseline: 1.3862x; 1.0610x over previous
import functools

import jax
import jax.numpy as jnp
from jax import lax
from jax.experimental import pallas as pl
from jax.experimental.pallas import tpu as pltpu
from jax.experimental.pallas import tpu_sc as plsc

DIM = 64
EPS = 1e-05

NUM_CORES = 2
NUM_SUBCORES = 16
NW = NUM_CORES * NUM_SUBCORES

CHUNK = 128


def _make_gather(batch: int):
  assert batch % (NW * CHUNK) == 0
  b_per_w = batch // NW
  n_chunks = b_per_w // CHUNK
  mesh = plsc.VectorSubcoreMesh(
      core_axis_name="c", subcore_axis_name="s",
      num_cores=NUM_CORES, num_subcores=NUM_SUBCORES)

  @functools.partial(
      pl.kernel,
      out_type=jax.ShapeDtypeStruct((batch, 2 * DIM), jnp.float32),
      mesh=mesh,
      scratch_types=[
          pltpu.VMEM((b_per_w,), jnp.int32),
          pltpu.VMEM((CHUNK, 2 * DIM), jnp.float32),
          pltpu.SemaphoreType.DMA,
      ],
  )
  def gather_kernel(idx_hbm, table_hbm, out_hbm, idx_v, rows_v, sem):
    wid = lax.axis_index("s") * NUM_CORES + lax.axis_index("c")
    base = wid * b_per_w
    pltpu.sync_copy(idx_hbm.at[pl.ds(base, b_per_w)], idx_v)

    @pl.loop(0, n_chunks)
    def _chunk(c):
      off = c * CHUNK
      pltpu.async_copy(
          table_hbm.at[idx_v.at[pl.ds(off, CHUNK)]], rows_v, sem).wait()
      pltpu.sync_copy(rows_v, out_hbm.at[pl.ds(base + off, CHUNK)])

  return gather_kernel


def _ln_body(x_ref, pm_ref, s_ref, b_ref, o_ref):
  g, seq, _ = o_ref.shape
  x = x_ref[...].reshape(g, seq, 2 * DIM)
  pm = pm_ref[...].reshape(g, seq, 1)
  xsel = jnp.where(pm > 0.5, x[:, :, DIM:], x[:, :, :DIM])
  mean = jnp.mean(xsel, axis=-1, keepdims=True)
  var = jnp.mean(jnp.square(xsel - mean), axis=-1, keepdims=True)
  inv = s_ref[...] * lax.rsqrt(var + EPS)
  o_ref[...] = xsel * inv + (b_ref[...] - mean * inv)


def _make_ln(groups: int, seq: int, gblk: int):
  assert groups % gblk == 0
  rblk = gblk * seq
  return pl.pallas_call(
      _ln_body,
      grid=(groups // gblk,),
      in_specs=[
          pl.BlockSpec((rblk, 2 * DIM), lambda i: (i, 0)),
          pl.BlockSpec((gblk, seq), lambda i: (i, 0)),
          pl.BlockSpec((1, 1, DIM), lambda i: (0, 0, 0)),
          pl.BlockSpec((1, 1, DIM), lambda i: (0, 0, 0)),
      ],
      out_specs=pl.BlockSpec((gblk, seq, DIM), lambda i: (i, 0, 0)),
      out_shape=jax.ShapeDtypeStruct((groups, seq, DIM), jnp.float32),
  )


def kernel(inputs, emb_weight, ln_scale, ln_bias):
  groups, seq = inputs.shape
  idx = jnp.asarray(inputs, jnp.int32).reshape(-1)
  table_pad = jnp.pad(emb_weight, ((0, 0), (0, DIM)))
  packed = _make_gather(groups * seq)(idx, table_pad)
  pmask = jnp.zeros((groups, seq), jnp.float32)
  return _make_ln(groups, seq, 64)(
      packed, pmask, ln_scale.reshape(1, 1, DIM), ln_bias.reshape(1, 1, DIM))


# speedup vs baseline: 1.4043x; 1.0130x over previous
import functools

import jax
import jax.numpy as jnp
from jax import lax
from jax.experimental import pallas as pl
from jax.experimental.pallas import tpu as pltpu
from jax.experimental.pallas import tpu_sc as plsc

DIM = 64
EPS = 1e-05

NUM_CORES = 2
NUM_SUBCORES = 16
NW = NUM_CORES * NUM_SUBCORES

CHUNK = 128


def _make_gather(batch: int):
  assert batch % (NW * CHUNK) == 0
  b_per_w = batch // NW
  n_chunks = b_per_w // CHUNK
  mesh = plsc.VectorSubcoreMesh(
      core_axis_name="c", subcore_axis_name="s",
      num_cores=NUM_CORES, num_subcores=NUM_SUBCORES)

  @functools.partial(
      pl.kernel,
      out_type=jax.ShapeDtypeStruct((batch, 2 * DIM), jnp.float32),
      mesh=mesh,
      scratch_types=[
          pltpu.VMEM((b_per_w,), jnp.int32),
          pltpu.VMEM((CHUNK, 2 * DIM), jnp.float32),
          pltpu.SemaphoreType.DMA,
      ],
  )
  def gather_kernel(idx_hbm, table_hbm, out_hbm, idx_v, rows_v, sem):
    wid = lax.axis_index("s") * NUM_CORES + lax.axis_index("c")
    base = wid * b_per_w
    pltpu.sync_copy(idx_hbm.at[pl.ds(base, b_per_w)], idx_v)

    @pl.loop(0, n_chunks)
    def _chunk(c):
      off = c * CHUNK
      pltpu.async_copy(
          table_hbm.at[idx_v.at[pl.ds(off, CHUNK)]], rows_v, sem).wait()
      pltpu.sync_copy(rows_v, out_hbm.at[pl.ds(base + off, CHUNK)])

  return gather_kernel


def _ln_body(x_ref, pm_ref, s_ref, b_ref, o_ref):
  g, seq, _ = o_ref.shape
  x = x_ref[...].reshape(g, seq, 2 * DIM)
  pm = pm_ref[...].reshape(g, seq, 1)
  xsel = jnp.where(pm > 0.5, x[:, :, DIM:], x[:, :, :DIM])
  mean = jnp.mean(xsel, axis=-1, keepdims=True)
  var = jnp.mean(jnp.square(xsel - mean), axis=-1, keepdims=True)
  inv = s_ref[...] * lax.rsqrt(var + EPS)
  o_ref[...] = xsel * inv + (b_ref[...] - mean * inv)


def _make_ln(groups: int, seq: int, gblk: int):
  assert groups % gblk == 0
  rblk = gblk * seq
  return pl.pallas_call(
      _ln_body,
      grid=(groups // gblk,),
      in_specs=[
          pl.BlockSpec((rblk, 2 * DIM), lambda i: (i, 0)),
          pl.BlockSpec((gblk, seq), lambda i: (i, 0)),
          pl.BlockSpec((1, 1, DIM), lambda i: (0, 0, 0)),
          pl.BlockSpec((1, 1, DIM), lambda i: (0, 0, 0)),
      ],
      out_specs=pl.BlockSpec((gblk, seq, DIM), lambda i: (i, 0, 0)),
      out_shape=jax.ShapeDtypeStruct((groups, seq, DIM), jnp.float32),
  )


def kernel(inputs, emb_weight, ln_scale, ln_bias):
  groups, seq = inputs.shape
  idx = jnp.asarray(inputs, jnp.int32).reshape(-1)
  table_pad = jnp.pad(emb_weight, ((0, 0), (0, DIM)))
  half = groups // 2
  gather = _make_gather(half * seq)
  ln = _make_ln(half, seq, 64)
  scale = ln_scale.reshape(1, 1, DIM)
  bias = ln_bias.reshape(1, 1, DIM)
  pmask = jnp.zeros((half, seq), jnp.float32)
  outs = []
  for h in range(2):
    packed = gather(idx[h * half * seq:(h + 1) * half * seq], table_pad)
    outs.append(ln(packed, pmask, scale, bias))
  return jnp.concatenate(outs, axis=0)
